# baseline (speedup 1.0000x reference)
import functools

import numpy as np
import jax
import jax.numpy as jnp
from jax.experimental import pallas as pl
from jax.experimental.pallas import tpu as pltpu
from jax.experimental.pallas import tpu_sc as plsc

S, D, E, K = 2048, 768, 8, 2
HR, HE = 768, 1536
DC = 2 * D
TS = 256
NT = S // TS
GT = 256
NP = S * K
NGT = NP // GT + E
PM = NGT * GT
OUT2 = K * S + S


def _pos_encoding():
    pos = np.arange(S)[:, None].astype(np.float32)
    div = np.exp(np.arange(0, D, 2).astype(np.float32) * (-np.log(10000.0) / D))
    pe = np.zeros((S, D), dtype=np.float32)
    pe[:, 0::2] = np.sin(pos * div)
    pe[:, 1::2] = np.cos(pos * div)
    return jnp.asarray(pe)


def _ln(x, g, b):
    m = jnp.mean(x, axis=-1, keepdims=True)
    v = jnp.mean((x - m) ** 2, axis=-1, keepdims=True)
    return (x - m) * jax.lax.rsqrt(v + 1e-5) * g + b


def _dot(a, b):
    return jnp.dot(a, b, preferred_element_type=jnp.float32)


def _router_kernel(x_ref, pe_ref, g_ref, b_ref, w1_ref, b1_ref, w2_ref, b2_ref,
                   t_ref, xp_ref, ti_ref, tw_ref):
    xp = x_ref[...] + pe_ref[...]
    xp_ref[...] = xp
    h = _ln(xp, g_ref[...], b_ref[...])
    a = jax.nn.gelu(_dot(h, w1_ref[...]) + b1_ref[...])
    logits = _dot(a, w2_ref[...]) + b2_ref[...]
    l = logits / t_ref[0, 0]
    m = jnp.max(l, axis=-1, keepdims=True)
    p = jnp.exp(l - m)
    probs = p / jnp.sum(p, axis=-1, keepdims=True)
    iota = jax.lax.broadcasted_iota(jnp.int32, (TS, E), 1)
    i1 = jnp.argmax(probs, axis=-1)[:, None]
    m1 = jnp.max(probs, axis=-1, keepdims=True)
    probs2 = jnp.where(iota == i1, -jnp.inf, probs)
    i2 = jnp.argmax(probs2, axis=-1)[:, None]
    m2 = jnp.max(probs2, axis=-1, keepdims=True)
    s = m1 + m2 + 1e-9
    ti_ref[...] = jnp.concatenate([i1, i2], axis=1)
    tw_ref[...] = jnp.concatenate([m1 / s, m2 / s], axis=1)


def _grouped_ffn_kernel(te_ref, gx_ref, wr_ref, w1_ref, b1_ref, w2_ref, b2_ref,
                        y_ref):
    a = jax.nn.gelu(_dot(gx_ref[...], w1_ref[0]) + b1_ref[0])
    y = _dot(a, w2_ref[0]) + b2_ref[0]
    y_ref[...] = y * wr_ref[...]


def _combiner_kernel(ca_ref, cb_ref, xp_ref, cg_ref, cbb_ref, w1_ref, b1_ref,
                     w2_ref, b2_ref, og_ref, ob_ref, out_ref):
    comb = ca_ref[...].astype(jnp.float32) + cb_ref[...].astype(jnp.float32)
    ch = _ln(comb, cg_ref[...], cbb_ref[...])
    a = jax.nn.gelu(_dot(ch, w1_ref[...]) + b1_ref[...])
    c = _dot(a, w2_ref[...]) + b2_ref[...]
    out_ref[...] = _ln(xp_ref[...] + c, og_ref[...], ob_ref[...])


_SC_MESH = plsc.VectorSubcoreMesh(core_axis_name="core",
                                  subcore_axis_name="subcore")
_NW = 32
_BW = PM // _NW
_CH = 96
assert _BW % _CH == 0 and PM % _NW == 0


def _sc_gather(xpf, rows):
    @functools.partial(
        pl.kernel, mesh=_SC_MESH,
        out_type=jax.ShapeDtypeStruct((PM, D), jnp.float32),
        scratch_types=[pltpu.VMEM((_CH,), jnp.int32),
                       pltpu.VMEM((_CH, D), jnp.float32),
                       pltpu.SemaphoreType.DMA],
    )
    def k(x_hbm, i_hbm, o_hbm, idx_v, rows_v, sem):
        wid = jax.lax.axis_index("subcore") * 2 + jax.lax.axis_index("core")
        base = wid * _BW

        @pl.loop(0, _BW // _CH)
        def _(c):
            off = base + c * _CH
            pltpu.sync_copy(i_hbm.at[pl.ds(off, _CH)], idx_v)
            pltpu.async_copy(x_hbm.at[idx_v], rows_v, sem).wait()
            pltpu.sync_copy(rows_v, o_hbm.at[pl.ds(off, _CH)])

    return k(xpf, rows)


def _sc_scatter(y, dest):
    @functools.partial(
        pl.kernel, mesh=_SC_MESH,
        out_type=jax.ShapeDtypeStruct((OUT2, D), jnp.float32),
        scratch_types=[pltpu.VMEM((_CH,), jnp.int32),
                       pltpu.VMEM((_CH, D), jnp.float32),
                       pltpu.SemaphoreType.DMA],
    )
    def k(y_hbm, i_hbm, o_hbm, idx_v, rows_v, sem):
        wid = jax.lax.axis_index("subcore") * 2 + jax.lax.axis_index("core")
        base = wid * _BW

        @pl.loop(0, _BW // _CH)
        def _(c):
            off = base + c * _CH
            pltpu.sync_copy(i_hbm.at[pl.ds(off, _CH)], idx_v)
            pltpu.sync_copy(y_hbm.at[pl.ds(off, _CH)], rows_v)
            pltpu.async_copy(rows_v, o_hbm.at[idx_v], sem).wait()

    return k(y, dest)


def kernel(x, rln_g, rln_b, rW1, rb1, rW2, rb2, temp, eW1, eb1, eW2, eb2,
           cln_g, cln_b, cW1, cb1, cW2, cb2, oln_g, oln_b):
    x2 = x.reshape(S, D)
    pe = _pos_encoding()
    row = lambda v: v.reshape(1, -1)

    xp, topi, topw = pl.pallas_call(
        _router_kernel,
        grid=(NT,),
        in_specs=[
            pl.BlockSpec((TS, D), lambda i: (i, 0)),
            pl.BlockSpec((TS, D), lambda i: (i, 0)),
            pl.BlockSpec((1, D), lambda i: (0, 0)),
            pl.BlockSpec((1, D), lambda i: (0, 0)),
            pl.BlockSpec((D, HR), lambda i: (0, 0)),
            pl.BlockSpec((1, HR), lambda i: (0, 0)),
            pl.BlockSpec((HR, E), lambda i: (0, 0)),
            pl.BlockSpec((1, E), lambda i: (0, 0)),
            pl.BlockSpec((1, 1), lambda i: (0, 0)),
        ],
        out_specs=[
            pl.BlockSpec((TS, D), lambda i: (i, 0)),
            pl.BlockSpec((TS, K), lambda i: (i, 0)),
            pl.BlockSpec((TS, K), lambda i: (i, 0)),
        ],
        out_shape=[
            jax.ShapeDtypeStruct((S, D), jnp.float32),
            jax.ShapeDtypeStruct((S, K), jnp.int32),
            jax.ShapeDtypeStruct((S, K), jnp.float32),
        ],
    )(x2, pe, row(rln_g), row(rln_b), rW1, row(rb1), rW2, row(rb2),
      temp.reshape(1, 1))

    e_p = topi.reshape(NP)
    w_p = topw.reshape(NP)
    t_p = jnp.arange(NP, dtype=jnp.int32) // K
    k_p = jnp.arange(NP, dtype=jnp.int32) % K
    oh = (e_p[:, None] == jnp.arange(E, dtype=jnp.int32)[None, :]
          ).astype(jnp.int32)
    csum = jnp.cumsum(oh, axis=0)
    counts = csum[-1]
    rank = jnp.take_along_axis(csum, e_p[:, None], axis=1)[:, 0] - 1
    pc = ((counts + GT - 1) // GT) * GT
    pstart = jnp.cumsum(pc) - pc
    ppos = pstart[e_p] + rank
    spread = jnp.arange(PM, dtype=jnp.int32) % S
    rows = spread.at[ppos].set(t_p)
    base = jnp.stack([K * S + spread, jnp.zeros((PM,), jnp.int32)], axis=1)
    packed = jnp.stack(
        [k_p * S + t_p, jax.lax.bitcast_convert_type(w_p, jnp.int32)], axis=1)
    plan = base.at[ppos].set(packed)
    dest = plan[:, 0]
    wrow = jax.lax.bitcast_convert_type(plan[:, 1], jnp.float32)
    tile_expert = jnp.minimum(
        jnp.searchsorted(jnp.cumsum(pc),
                         jnp.arange(NGT, dtype=jnp.int32) * GT, side="right"),
        E - 1).astype(jnp.int32)

    gx = _sc_gather(xp, rows)

    y = pl.pallas_call(
        _grouped_ffn_kernel,
        grid_spec=pltpu.PrefetchScalarGridSpec(
            num_scalar_prefetch=1,
            grid=(NGT,),
            in_specs=[
                pl.BlockSpec((GT, D), lambda i, te: (i, 0)),
                pl.BlockSpec((GT, 1), lambda i, te: (i, 0)),
                pl.BlockSpec((1, D, HE), lambda i, te: (te[i], 0, 0)),
                pl.BlockSpec((1, 1, HE), lambda i, te: (te[i], 0, 0)),
                pl.BlockSpec((1, HE, D), lambda i, te: (te[i], 0, 0)),
                pl.BlockSpec((1, 1, D), lambda i, te: (te[i], 0, 0)),
            ],
            out_specs=pl.BlockSpec((GT, D), lambda i, te: (i, 0)),
        ),
        out_shape=jax.ShapeDtypeStruct((PM, D), jnp.float32),
    )(tile_expert, gx, wrow.reshape(PM, 1), eW1,
      eb1.reshape(E, 1, HE), eW2, eb2.reshape(E, 1, D))

    out2 = _sc_scatter(y, dest)

    out = pl.pallas_call(
        _combiner_kernel,
        grid=(NT,),
        in_specs=[
            pl.BlockSpec((TS, D), lambda i: (i, 0)),
            pl.BlockSpec((TS, D), lambda i: (i + NT, 0)),
            pl.BlockSpec((TS, D), lambda i: (i, 0)),
            pl.BlockSpec((1, D), lambda i: (0, 0)),
            pl.BlockSpec((1, D), lambda i: (0, 0)),
            pl.BlockSpec((D, DC), lambda i: (0, 0)),
            pl.BlockSpec((1, DC), lambda i: (0, 0)),
            pl.BlockSpec((DC, D), lambda i: (0, 0)),
            pl.BlockSpec((1, D), lambda i: (0, 0)),
            pl.BlockSpec((1, D), lambda i: (0, 0)),
            pl.BlockSpec((1, D), lambda i: (0, 0)),
        ],
        out_specs=pl.BlockSpec((TS, D), lambda i: (i, 0)),
        out_shape=jax.ShapeDtypeStruct((S, D), jnp.float32),
    )(out2, out2, xp, row(cln_g), row(cln_b), cW1, row(cb1), cW2, row(cb2),
      row(oln_g), row(oln_b))

    return out.reshape(1, S, D)


# speedup vs baseline: 1.0215x; 1.0215x over previous
import functools

import numpy as np
import jax
import jax.numpy as jnp
from jax.experimental import pallas as pl
from jax.experimental.pallas import tpu as pltpu
from jax.experimental.pallas import tpu_sc as plsc

S, D, E, K = 2048, 768, 8, 2
HR, HE = 768, 1536
DC = 2 * D
TS = 256
NT = S // TS
GT = 256
NP = S * K
NGT = NP // GT + E
PM = NGT * GT
OUT2 = K * S + S


def _pos_encoding():
    pos = np.arange(S)[:, None].astype(np.float32)
    div = np.exp(np.arange(0, D, 2).astype(np.float32) * (-np.log(10000.0) / D))
    pe = np.zeros((S, D), dtype=np.float32)
    pe[:, 0::2] = np.sin(pos * div)
    pe[:, 1::2] = np.cos(pos * div)
    return jnp.asarray(pe)


def _ln(x, g, b):
    m = jnp.mean(x, axis=-1, keepdims=True)
    v = jnp.mean((x - m) ** 2, axis=-1, keepdims=True)
    return (x - m) * jax.lax.rsqrt(v + 1e-5) * g + b


def _dot(a, b):
    return jnp.dot(a, b, preferred_element_type=jnp.float32)


def _router_kernel(x_ref, pe_ref, g_ref, b_ref, w1_ref, b1_ref, w2_ref, b2_ref,
                   t_ref, xp_ref, ti_ref, tw_ref):
    xp = x_ref[...] + pe_ref[...]
    xp_ref[...] = xp
    h = _ln(xp, g_ref[...], b_ref[...])
    a = jax.nn.gelu(_dot(h, w1_ref[...]) + b1_ref[...])
    logits = _dot(a, w2_ref[...]) + b2_ref[...]
    l = logits / t_ref[0, 0]
    m = jnp.max(l, axis=-1, keepdims=True)
    p = jnp.exp(l - m)
    probs = p / jnp.sum(p, axis=-1, keepdims=True)
    iota = jax.lax.broadcasted_iota(jnp.int32, (TS, E), 1)
    i1 = jnp.argmax(probs, axis=-1)[:, None]
    m1 = jnp.max(probs, axis=-1, keepdims=True)
    probs2 = jnp.where(iota == i1, -jnp.inf, probs)
    i2 = jnp.argmax(probs2, axis=-1)[:, None]
    m2 = jnp.max(probs2, axis=-1, keepdims=True)
    s = m1 + m2 + 1e-9
    ti_ref[...] = jnp.concatenate([i1, i2], axis=1)
    tw_ref[...] = jnp.concatenate([m1 / s, m2 / s], axis=1)


def _grouped_ffn_kernel(te_ref, gx_ref, wr_ref, w1_ref, b1_ref, w2_ref, b2_ref,
                        y_ref):
    a = jax.nn.gelu(_dot(gx_ref[...], w1_ref[0]) + b1_ref[0])
    y = _dot(a, w2_ref[0]) + b2_ref[0]
    y_ref[...] = y * wr_ref[...]


def _combiner_kernel(ca_ref, cb_ref, xp_ref, cg_ref, cbb_ref, w1_ref, b1_ref,
                     w2_ref, b2_ref, og_ref, ob_ref, out_ref):
    comb = ca_ref[...].astype(jnp.float32) + cb_ref[...].astype(jnp.float32)
    ch = _ln(comb, cg_ref[...], cbb_ref[...])
    a = jax.nn.gelu(_dot(ch, w1_ref[...]) + b1_ref[...])
    c = _dot(a, w2_ref[...]) + b2_ref[...]
    out_ref[...] = _ln(xp_ref[...] + c, og_ref[...], ob_ref[...])


_SC_MESH = plsc.VectorSubcoreMesh(core_axis_name="core",
                                  subcore_axis_name="subcore")
_NW = 32
_BW = PM // _NW
_CH = 96
assert _BW % _CH == 0 and PM % _NW == 0


def _sc_gather(xpf, rows):
    @functools.partial(
        pl.kernel, mesh=_SC_MESH,
        out_type=jax.ShapeDtypeStruct((PM, D), jnp.float32),
        scratch_types=[pltpu.VMEM((_CH,), jnp.int32),
                       pltpu.VMEM((_CH, D), jnp.float32),
                       pltpu.SemaphoreType.DMA],
    )
    def k(x_hbm, i_hbm, o_hbm, idx_v, rows_v, sem):
        wid = jax.lax.axis_index("subcore") * 2 + jax.lax.axis_index("core")
        base = wid * _BW

        @pl.loop(0, _BW // _CH)
        def _(c):
            off = base + c * _CH
            pltpu.sync_copy(i_hbm.at[pl.ds(off, _CH)], idx_v)
            pltpu.async_copy(x_hbm.at[idx_v], rows_v, sem).wait()
            pltpu.sync_copy(rows_v, o_hbm.at[pl.ds(off, _CH)])

    return k(xpf, rows)


def _sc_scatter(y, dest):
    @functools.partial(
        pl.kernel, mesh=_SC_MESH,
        out_type=jax.ShapeDtypeStruct((OUT2, D), jnp.float32),
        scratch_types=[pltpu.VMEM((_CH,), jnp.int32),
                       pltpu.VMEM((_CH, D), jnp.float32),
                       pltpu.SemaphoreType.DMA],
    )
    def k(y_hbm, i_hbm, o_hbm, idx_v, rows_v, sem):
        wid = jax.lax.axis_index("subcore") * 2 + jax.lax.axis_index("core")
        base = wid * _BW

        @pl.loop(0, _BW // _CH)
        def _(c):
            off = base + c * _CH
            pltpu.sync_copy(i_hbm.at[pl.ds(off, _CH)], idx_v)
            pltpu.sync_copy(y_hbm.at[pl.ds(off, _CH)], rows_v)
            pltpu.async_copy(rows_v, o_hbm.at[idx_v], sem).wait()

    return k(y, dest)


def kernel(x, rln_g, rln_b, rW1, rb1, rW2, rb2, temp, eW1, eb1, eW2, eb2,
           cln_g, cln_b, cW1, cb1, cW2, cb2, oln_g, oln_b):
    x2 = x.reshape(S, D)
    pe = _pos_encoding()
    row = lambda v: v.reshape(1, -1)

    xp, topi, topw = pl.pallas_call(
        _router_kernel,
        grid=(NT,),
        in_specs=[
            pl.BlockSpec((TS, D), lambda i: (i, 0)),
            pl.BlockSpec((TS, D), lambda i: (i, 0)),
            pl.BlockSpec((1, D), lambda i: (0, 0)),
            pl.BlockSpec((1, D), lambda i: (0, 0)),
            pl.BlockSpec((D, HR), lambda i: (0, 0)),
            pl.BlockSpec((1, HR), lambda i: (0, 0)),
            pl.BlockSpec((HR, E), lambda i: (0, 0)),
            pl.BlockSpec((1, E), lambda i: (0, 0)),
            pl.BlockSpec((1, 1), lambda i: (0, 0)),
        ],
        out_specs=[
            pl.BlockSpec((TS, D), lambda i: (i, 0)),
            pl.BlockSpec((TS, K), lambda i: (i, 0)),
            pl.BlockSpec((TS, K), lambda i: (i, 0)),
        ],
        out_shape=[
            jax.ShapeDtypeStruct((S, D), jnp.float32),
            jax.ShapeDtypeStruct((S, K), jnp.int32),
            jax.ShapeDtypeStruct((S, K), jnp.float32),
        ],
    )(x2, pe, row(rln_g), row(rln_b), rW1, row(rb1), rW2, row(rb2),
      temp.reshape(1, 1))

    e_p = topi.reshape(NP)
    w_p = topw.reshape(NP)
    t_p = jnp.arange(NP, dtype=jnp.int32) // K
    k_p = jnp.arange(NP, dtype=jnp.int32) % K
    oh = (e_p[:, None] == jnp.arange(E, dtype=jnp.int32)[None, :]
          ).astype(jnp.int32)
    csum = jnp.cumsum(oh, axis=0)
    counts = csum[-1]
    rank = jnp.take_along_axis(csum, e_p[:, None], axis=1)[:, 0] - 1
    pc = ((counts + GT - 1) // GT) * GT
    pstart = jnp.cumsum(pc) - pc
    ppos = pstart[e_p] + rank
    spread = jnp.arange(PM, dtype=jnp.int32) % S
    zero = jnp.zeros((PM,), jnp.int32)
    base = jnp.stack([spread, K * S + spread, zero, zero], axis=1)
    packed = jnp.stack(
        [t_p, k_p * S + t_p, jax.lax.bitcast_convert_type(w_p, jnp.int32),
         jnp.zeros_like(t_p)], axis=1)
    plan = base.at[ppos].set(packed)
    rows = plan[:, 0]
    dest = plan[:, 1]
    wrow = jax.lax.bitcast_convert_type(plan[:, 2], jnp.float32)
    tile_expert = jnp.minimum(
        jnp.searchsorted(jnp.cumsum(pc),
                         jnp.arange(NGT, dtype=jnp.int32) * GT, side="right"),
        E - 1).astype(jnp.int32)

    gx = _sc_gather(xp, rows)

    y = pl.pallas_call(
        _grouped_ffn_kernel,
        grid_spec=pltpu.PrefetchScalarGridSpec(
            num_scalar_prefetch=1,
            grid=(NGT,),
            in_specs=[
                pl.BlockSpec((GT, D), lambda i, te: (i, 0)),
                pl.BlockSpec((GT, 1), lambda i, te: (i, 0)),
                pl.BlockSpec((1, D, HE), lambda i, te: (te[i], 0, 0)),
                pl.BlockSpec((1, 1, HE), lambda i, te: (te[i], 0, 0)),
                pl.BlockSpec((1, HE, D), lambda i, te: (te[i], 0, 0)),
                pl.BlockSpec((1, 1, D), lambda i, te: (te[i], 0, 0)),
            ],
            out_specs=pl.BlockSpec((GT, D), lambda i, te: (i, 0)),
        ),
        out_shape=jax.ShapeDtypeStruct((PM, D), jnp.float32),
    )(tile_expert, gx, wrow.reshape(PM, 1), eW1,
      eb1.reshape(E, 1, HE), eW2, eb2.reshape(E, 1, D))

    out2 = _sc_scatter(y, dest)

    out = pl.pallas_call(
        _combiner_kernel,
        grid=(NT,),
        in_specs=[
            pl.BlockSpec((TS, D), lambda i: (i, 0)),
            pl.BlockSpec((TS, D), lambda i: (i + NT, 0)),
            pl.BlockSpec((TS, D), lambda i: (i, 0)),
            pl.BlockSpec((1, D), lambda i: (0, 0)),
            pl.BlockSpec((1, D), lambda i: (0, 0)),
            pl.BlockSpec((D, DC), lambda i: (0, 0)),
            pl.BlockSpec((1, DC), lambda i: (0, 0)),
            pl.BlockSpec((DC, D), lambda i: (0, 0)),
            pl.BlockSpec((1, D), lambda i: (0, 0)),
            pl.BlockSpec((1, D), lambda i: (0, 0)),
            pl.BlockSpec((1, D), lambda i: (0, 0)),
        ],
        out_specs=pl.BlockSpec((TS, D), lambda i: (i, 0)),
        out_shape=jax.ShapeDtypeStruct((S, D), jnp.float32),
    )(out2, out2, xp, row(cln_g), row(cln_b), cW1, row(cb1), cW2, row(cb2),
      row(oln_g), row(oln_b))

    return out.reshape(1, S, D)


# speedup vs baseline: 1.0334x; 1.0116x over previous
import functools

import numpy as np
import jax
import jax.numpy as jnp
from jax.experimental import pallas as pl
from jax.experimental.pallas import tpu as pltpu
from jax.experimental.pallas import tpu_sc as plsc

S, D, E, K = 2048, 768, 8, 2
HR, HE = 768, 1536
DC = 2 * D
TS = 256
NT = S // TS
GT = 256
NP = S * K
NGT = NP // GT + E
PM = NGT * GT


def _pos_encoding():
    pos = np.arange(S)[:, None].astype(np.float32)
    div = np.exp(np.arange(0, D, 2).astype(np.float32) * (-np.log(10000.0) / D))
    pe = np.zeros((S, D), dtype=np.float32)
    pe[:, 0::2] = np.sin(pos * div)
    pe[:, 1::2] = np.cos(pos * div)
    return jnp.asarray(pe)


def _ln(x, g, b):
    m = jnp.mean(x, axis=-1, keepdims=True)
    v = jnp.mean((x - m) ** 2, axis=-1, keepdims=True)
    return (x - m) * jax.lax.rsqrt(v + 1e-5) * g + b


def _dot(a, b):
    return jnp.dot(a, b, preferred_element_type=jnp.float32)


def _router_kernel(x_ref, pe_ref, g_ref, b_ref, w1_ref, b1_ref, w2_ref, b2_ref,
                   t_ref, xp_ref, ti_ref, tw_ref):
    xp = x_ref[...] + pe_ref[...]
    xp_ref[...] = xp
    h = _ln(xp, g_ref[...], b_ref[...])
    a = jax.nn.gelu(_dot(h, w1_ref[...]) + b1_ref[...])
    logits = _dot(a, w2_ref[...]) + b2_ref[...]
    l = logits / t_ref[0, 0]
    m = jnp.max(l, axis=-1, keepdims=True)
    p = jnp.exp(l - m)
    probs = p / jnp.sum(p, axis=-1, keepdims=True)
    iota = jax.lax.broadcasted_iota(jnp.int32, (TS, E), 1)
    i1 = jnp.argmax(probs, axis=-1)[:, None]
    m1 = jnp.max(probs, axis=-1, keepdims=True)
    probs2 = jnp.where(iota == i1, -jnp.inf, probs)
    i2 = jnp.argmax(probs2, axis=-1)[:, None]
    m2 = jnp.max(probs2, axis=-1, keepdims=True)
    s = m1 + m2 + 1e-9
    ti_ref[...] = jnp.concatenate([i1, i2], axis=1)
    tw_ref[...] = jnp.concatenate([m1 / s, m2 / s], axis=1)


def _grouped_ffn_kernel(te_ref, gx_ref, wr_ref, w1_ref, b1_ref, w2_ref, b2_ref,
                        y_ref):
    a = jax.nn.gelu(_dot(gx_ref[...], w1_ref[0]) + b1_ref[0])
    y = _dot(a, w2_ref[0]) + b2_ref[0]
    y_ref[...] = y * wr_ref[...]


def _combiner_kernel(ca_ref, cb_ref, xp_ref, cg_ref, cbb_ref, w1_ref, b1_ref,
                     w2_ref, b2_ref, og_ref, ob_ref, out_ref):
    comb = ca_ref[...].astype(jnp.float32) + cb_ref[...].astype(jnp.float32)
    ch = _ln(comb, cg_ref[...], cbb_ref[...])
    a = jax.nn.gelu(_dot(ch, w1_ref[...]) + b1_ref[...])
    c = _dot(a, w2_ref[...]) + b2_ref[...]
    out_ref[...] = _ln(xp_ref[...] + c, og_ref[...], ob_ref[...])


_SC_MESH = plsc.VectorSubcoreMesh(core_axis_name="core",
                                  subcore_axis_name="subcore")
_NW = 32


def _sc_gather(src_arr, idx, n_rows, ch):
    assert n_rows % _NW == 0 and (n_rows // _NW) % ch == 0

    @functools.partial(
        pl.kernel, mesh=_SC_MESH,
        out_type=jax.ShapeDtypeStruct((n_rows, D), jnp.float32),
        scratch_types=[pltpu.VMEM((ch,), jnp.int32),
                       pltpu.VMEM((ch, D), jnp.float32),
                       pltpu.SemaphoreType.DMA],
    )
    def k(x_hbm, i_hbm, o_hbm, idx_v, rows_v, sem):
        wid = jax.lax.axis_index("subcore") * 2 + jax.lax.axis_index("core")
        bw = n_rows // _NW
        base = wid * bw

        @pl.loop(0, bw // ch)
        def _(c):
            off = base + c * ch
            pltpu.sync_copy(i_hbm.at[pl.ds(off, ch)], idx_v)
            pltpu.async_copy(x_hbm.at[idx_v], rows_v, sem).wait()
            pltpu.sync_copy(rows_v, o_hbm.at[pl.ds(off, ch)])

    return k(src_arr, idx)


def kernel(x, rln_g, rln_b, rW1, rb1, rW2, rb2, temp, eW1, eb1, eW2, eb2,
           cln_g, cln_b, cW1, cb1, cW2, cb2, oln_g, oln_b):
    x2 = x.reshape(S, D)
    pe = _pos_encoding()
    row = lambda v: v.reshape(1, -1)

    xp, topi, topw = pl.pallas_call(
        _router_kernel,
        grid=(NT,),
        in_specs=[
            pl.BlockSpec((TS, D), lambda i: (i, 0)),
            pl.BlockSpec((TS, D), lambda i: (i, 0)),
            pl.BlockSpec((1, D), lambda i: (0, 0)),
            pl.BlockSpec((1, D), lambda i: (0, 0)),
            pl.BlockSpec((D, HR), lambda i: (0, 0)),
            pl.BlockSpec((1, HR), lambda i: (0, 0)),
            pl.BlockSpec((HR, E), lambda i: (0, 0)),
            pl.BlockSpec((1, E), lambda i: (0, 0)),
            pl.BlockSpec((1, 1), lambda i: (0, 0)),
        ],
        out_specs=[
            pl.BlockSpec((TS, D), lambda i: (i, 0)),
            pl.BlockSpec((TS, K), lambda i: (i, 0)),
            pl.BlockSpec((TS, K), lambda i: (i, 0)),
        ],
        out_shape=[
            jax.ShapeDtypeStruct((S, D), jnp.float32),
            jax.ShapeDtypeStruct((S, K), jnp.int32),
            jax.ShapeDtypeStruct((S, K), jnp.float32),
        ],
    )(x2, pe, row(rln_g), row(rln_b), rW1, row(rb1), rW2, row(rb2),
      temp.reshape(1, 1))

    e_p = topi.reshape(NP)
    w_p = topw.reshape(NP)
    t_p = jnp.arange(NP, dtype=jnp.int32) // K
    k_p = jnp.arange(NP, dtype=jnp.int32) % K
    oh = (e_p[:, None] == jnp.arange(E, dtype=jnp.int32)[None, :]
          ).astype(jnp.int32)
    csum = jnp.cumsum(oh, axis=0)
    counts = csum[-1]
    rank = jnp.take_along_axis(csum, e_p[:, None], axis=1)[:, 0] - 1
    pc = ((counts + GT - 1) // GT) * GT
    pstart = jnp.cumsum(pc) - pc
    ppos = pstart[e_p] + rank
    spread = jnp.arange(PM, dtype=jnp.int32) % S
    base = jnp.stack([spread, jnp.zeros((PM,), jnp.int32)], axis=1)
    packed = jnp.stack(
        [t_p, jax.lax.bitcast_convert_type(w_p, jnp.int32)], axis=1)
    plan = base.at[ppos].set(packed)
    rows = plan[:, 0]
    wrow = jax.lax.bitcast_convert_type(plan[:, 1], jnp.float32)
    gpos = ppos.reshape(S, K).T.reshape(K * S)
    tile_expert = jnp.minimum(
        jnp.searchsorted(jnp.cumsum(pc),
                         jnp.arange(NGT, dtype=jnp.int32) * GT, side="right"),
        E - 1).astype(jnp.int32)

    gx = _sc_gather(xp, rows, PM, 96)

    y = pl.pallas_call(
        _grouped_ffn_kernel,
        grid_spec=pltpu.PrefetchScalarGridSpec(
            num_scalar_prefetch=1,
            grid=(NGT,),
            in_specs=[
                pl.BlockSpec((GT, D), lambda i, te: (i, 0)),
                pl.BlockSpec((GT, 1), lambda i, te: (i, 0)),
                pl.BlockSpec((1, D, HE), lambda i, te: (te[i], 0, 0)),
                pl.BlockSpec((1, 1, HE), lambda i, te: (te[i], 0, 0)),
                pl.BlockSpec((1, HE, D), lambda i, te: (te[i], 0, 0)),
                pl.BlockSpec((1, 1, D), lambda i, te: (te[i], 0, 0)),
            ],
            out_specs=pl.BlockSpec((GT, D), lambda i, te: (i, 0)),
        ),
        out_shape=jax.ShapeDtypeStruct((PM, D), jnp.float32),
    )(tile_expert, gx, wrow.reshape(PM, 1), eW1,
      eb1.reshape(E, 1, HE), eW2, eb2.reshape(E, 1, D))

    out2 = _sc_gather(y, gpos, K * S, 128)

    out = pl.pallas_call(
        _combiner_kernel,
        grid=(NT,),
        in_specs=[
            pl.BlockSpec((TS, D), lambda i: (i, 0)),
            pl.BlockSpec((TS, D), lambda i: (i + NT, 0)),
            pl.BlockSpec((TS, D), lambda i: (i, 0)),
            pl.BlockSpec((1, D), lambda i: (0, 0)),
            pl.BlockSpec((1, D), lambda i: (0, 0)),
            pl.BlockSpec((D, DC), lambda i: (0, 0)),
            pl.BlockSpec((1, DC), lambda i: (0, 0)),
            pl.BlockSpec((DC, D), lambda i: (0, 0)),
            pl.BlockSpec((1, D), lambda i: (0, 0)),
            pl.BlockSpec((1, D), lambda i: (0, 0)),
            pl.BlockSpec((1, D), lambda i: (0, 0)),
        ],
        out_specs=pl.BlockSpec((TS, D), lambda i: (i, 0)),
        out_shape=jax.ShapeDtypeStruct((S, D), jnp.float32),
    )(out2, out2, xp, row(cln_g), row(cln_b), cW1, row(cb1), cW2, row(cb2),
      row(oln_g), row(oln_b))

    return out.reshape(1, S, D)
